# Initial kernel scaffold; baseline (speedup 1.0000x reference)
import jax
import jax.numpy as jnp
from jax.experimental import pallas as pl


def kernel(batch, X, H, edge_index, edge_attr, te, e3_w, ln_w, ln_b, pm_w1, pm_b1, pm_w2, pm_b2, px_w1, px_b1, px_w2, px_b2):
    raise NotImplementedError("write your pallas kernel here")


# speedup vs baseline: 7.5220x; 7.5220x over previous
import functools

import jax
import jax.numpy as jnp
from jax import lax
from jax.experimental import pallas as pl
from jax.experimental.pallas import tpu as pltpu

N = 50000
E = 800000
K = 64
HD = 64
ED = 16
NB = 256
D3 = 3 * K

BN = 1000
NBLK_N = N // BN
BE = 2000
NBLK_E = E // BE


def _onehot(bvec, n_seg):
    return (bvec[:, None] == lax.broadcasted_iota(jnp.int32, (bvec.shape[0], n_seg), 1)).astype(jnp.float32)


def _dot(a, b):
    return lax.dot_general(a, b, (((1,), (0,)), ((), ())), preferred_element_type=jnp.float32)


def _dott(a, b):
    return lax.dot_general(a, b, (((0,), (0,)), ((), ())), preferred_element_type=jnp.float32)


def _k1a_body(xb, bb, ob):
    i = pl.program_id(0)
    oh = _onehot(bb[0, 0, :], NB)
    xpad = jnp.pad(xb[...], ((0, 0), (0, NB - D3)))
    col = lax.broadcasted_iota(jnp.int32, (BN, NB), 1)
    xext = jnp.where(col == D3, 1.0, xpad)
    p = _dott(oh, xext)

    @pl.when(i == 0)
    def _():
        ob[...] = p

    @pl.when(i > 0)
    def _():
        ob[...] = ob[...] + p


def _k1a(x2, b3):
    return pl.pallas_call(
        _k1a_body,
        grid=(NBLK_N,),
        in_specs=[
            pl.BlockSpec((BN, D3), lambda i: (i, 0)),
            pl.BlockSpec((1, 1, BN), lambda i: (i, 0, 0)),
        ],
        out_specs=pl.BlockSpec((NB, NB), lambda i: (0, 0)),
        out_shape=jax.ShapeDtypeStruct((NB, NB), jnp.float32),
    )(x2, b3)


def _k1b_body(xb, bb, sb, ob):
    i = pl.program_id(0)
    sums = sb[...]
    cnt = jnp.maximum(sums[:, D3:D3 + 1], 1.0)
    mean_x = sums[:, :D3] / cnt
    oh = _onehot(bb[0, 0, :], NB)
    meanb = _dot(oh, mean_x)
    xc = xb[...] - meanb
    nrm = jnp.sqrt(xc[:, 0:K] ** 2 + xc[:, K:2 * K] ** 2 + xc[:, 2 * K:3 * K] ** 2)
    p = _dott(oh, nrm)

    @pl.when(i == 0)
    def _():
        ob[...] = p

    @pl.when(i > 0)
    def _():
        ob[...] = ob[...] + p


def _k1b(x2, b3, sums):
    return pl.pallas_call(
        _k1b_body,
        grid=(NBLK_N,),
        in_specs=[
            pl.BlockSpec((BN, D3), lambda i: (i, 0)),
            pl.BlockSpec((1, 1, BN), lambda i: (i, 0, 0)),
            pl.BlockSpec((NB, NB), lambda i: (0, 0)),
        ],
        out_specs=pl.BlockSpec((NB, K), lambda i: (0, 0)),
        out_shape=jax.ShapeDtypeStruct((NB, K), jnp.float32),
    )(x2, b3, sums)


def _k1c_body(xb, hb, bb, sb, nb_, ab, wb, xo, gto, gso):
    sums = sb[...]
    cnt = jnp.maximum(sums[:, D3:D3 + 1], 1.0)
    mean_x = sums[:, :D3] / cnt
    mn = nb_[...] / cnt
    oh = _onehot(bb[0, 0, :], NB)
    meanb = _dot(oh, mean_x)
    mnb = _dot(oh, mn)
    xc = xb[...] - meanb
    denom = mnb + 1e-5
    den3 = jnp.concatenate([denom, denom, denom], axis=1)
    e3 = ab[2:3, :]
    e33 = jnp.concatenate([e3, e3, e3], axis=1)
    xo[...] = xc * e33 / den3
    h = hb[...]
    mu = jnp.mean(h, axis=1, keepdims=True)
    var = jnp.mean((h - mu) ** 2, axis=1, keepdims=True)
    hn = (h - mu) * lax.rsqrt(var + 1e-5) * ab[0:1, :] + ab[1:2, :]
    gto[...] = _dot(hn, wb[0:HD, :])
    gso[...] = _dot(hn, wb[HD:2 * HD, :])


def _k1c(x2, h, b3, sums, nsums, aux, w1):
    return pl.pallas_call(
        _k1c_body,
        grid=(NBLK_N,),
        in_specs=[
            pl.BlockSpec((BN, D3), lambda i: (i, 0)),
            pl.BlockSpec((BN, HD), lambda i: (i, 0)),
            pl.BlockSpec((1, 1, BN), lambda i: (i, 0, 0)),
            pl.BlockSpec((NB, NB), lambda i: (0, 0)),
            pl.BlockSpec((NB, K), lambda i: (0, 0)),
            pl.BlockSpec((8, K), lambda i: (0, 0)),
            pl.BlockSpec((2 * HD + ED + K + HD, HD), lambda i: (0, 0)),
        ],
        out_specs=[
            pl.BlockSpec((BN, D3), lambda i: (i, 0)),
            pl.BlockSpec((BN, HD), lambda i: (i, 0)),
            pl.BlockSpec((BN, HD), lambda i: (i, 0)),
        ],
        out_shape=[
            jax.ShapeDtypeStruct((N, D3), jnp.float32),
            jax.ShapeDtypeStruct((N, HD), jnp.float32),
            jax.ShapeDtypeStruct((N, HD), jnp.float32),
        ],
    )(x2, h, b3, sums, nsums, aux, w1)


def _silu(x):
    return x / (1.0 + jnp.exp(-x))


def _k3_body(pb, rb, eb, tb, w1, w2, xw1, xw2, ab, so):
    rd = rb[...]
    pre = (pb[...] + _dot(eb[...], w1[2 * HD:2 * HD + ED, :])
           + _dot(rd, w1[2 * HD + ED:2 * HD + ED + K, :])
           + _dot(tb[...], w1[2 * HD + ED + K:, :]) + ab[3:4, :])
    h1 = _silu(pre)
    m = _dot(h1, w2[...]) + ab[4:5, :]
    h2 = _silu(_dot(m, xw1[...]) + ab[5:6, :])
    w = jnp.clip(_dot(h2, xw2[...]) + ab[6:7, :], -10.0, 10.0)
    so[...] = w / (1.0 + jnp.sqrt(rd + 1e-8))


def _k3(partial, rd, ea, te, w1, w2, xw1, xw2, aux):
    return pl.pallas_call(
        _k3_body,
        grid=(NBLK_E,),
        in_specs=[
            pl.BlockSpec((BE, HD), lambda i: (i, 0)),
            pl.BlockSpec((BE, K), lambda i: (i, 0)),
            pl.BlockSpec((BE, ED), lambda i: (i, 0)),
            pl.BlockSpec((BE, HD), lambda i: (i, 0)),
            pl.BlockSpec((2 * HD + ED + K + HD, HD), lambda i: (0, 0)),
            pl.BlockSpec((HD, HD), lambda i: (0, 0)),
            pl.BlockSpec((HD, HD), lambda i: (0, 0)),
            pl.BlockSpec((HD, K), lambda i: (0, 0)),
            pl.BlockSpec((8, K), lambda i: (0, 0)),
        ],
        out_specs=pl.BlockSpec((BE, K), lambda i: (i, 0)),
        out_shape=jax.ShapeDtypeStruct((E, K), jnp.float32),
    )(partial, rd, ea, te, w1, w2, xw1, xw2, aux)


def kernel(batch, X, H, edge_index, edge_attr, te, e3_w, ln_w, ln_b,
           pm_w1, pm_b1, pm_w2, pm_b2, px_w1, px_b1, px_w2, px_b2):
    x2 = X.reshape(N, D3)
    b32 = batch.astype(jnp.int32)
    b3 = b32.reshape(NBLK_N, 1, BN)
    src = edge_index[0].astype(jnp.int32)
    tgt = edge_index[1].astype(jnp.int32)

    aux = jnp.stack([ln_w, ln_b, e3_w.reshape(K), pm_b1, pm_b2,
                     px_b1, px_b2, jnp.zeros((K,), jnp.float32)], axis=0)

    sums = _k1a(x2, b3)
    nsums = _k1b(x2, b3, sums)
    xn, gt, gs = _k1c(x2, H, b3, sums, nsums, aux, pm_w1)

    rel = xn[src] - xn[tgt]
    rd = rel[:, 0:K] ** 2 + rel[:, K:2 * K] ** 2 + rel[:, 2 * K:3 * K] ** 2
    partial = gt[tgt] + gs[src]

    scale = _k3(partial, rd, edge_attr, te, pm_w1, pm_w2, px_w1, px_w2, aux)

    upd = rel * jnp.concatenate([scale, scale, scale], axis=1)
    out2 = xn + jax.ops.segment_sum(upd, tgt, num_segments=N)
    return out2.reshape(N, 3, K)


# speedup vs baseline: 15.9730x; 2.1235x over previous
import functools

import jax
import jax.numpy as jnp
from jax import lax
from jax.experimental import pallas as pl
from jax.experimental.pallas import tpu as pltpu
from jax.experimental.pallas import tpu_sc as plsc

N = 50000
E = 800000
K = 64
HD = 64
ED = 16
NB = 256
D3 = 3 * K

BN = 1000
NBLK_N = N // BN
BE = 2000
NBLK_E = E // BE


def _onehot(bvec, n_seg):
    return (bvec[:, None] == lax.broadcasted_iota(jnp.int32, (bvec.shape[0], n_seg), 1)).astype(jnp.float32)


def _dot(a, b):
    return lax.dot_general(a, b, (((1,), (0,)), ((), ())), preferred_element_type=jnp.float32)


def _dott(a, b):
    return lax.dot_general(a, b, (((0,), (0,)), ((), ())), preferred_element_type=jnp.float32)


def _k1a_body(xb, bb, ob):
    i = pl.program_id(0)
    oh = _onehot(bb[0, 0, :], NB)
    xpad = jnp.pad(xb[...], ((0, 0), (0, NB - D3)))
    col = lax.broadcasted_iota(jnp.int32, (BN, NB), 1)
    xext = jnp.where(col == D3, 1.0, xpad)
    p = _dott(oh, xext)

    @pl.when(i == 0)
    def _():
        ob[...] = p

    @pl.when(i > 0)
    def _():
        ob[...] = ob[...] + p


def _k1a(x2, b3):
    return pl.pallas_call(
        _k1a_body,
        grid=(NBLK_N,),
        in_specs=[
            pl.BlockSpec((BN, D3), lambda i: (i, 0)),
            pl.BlockSpec((1, 1, BN), lambda i: (i, 0, 0)),
        ],
        out_specs=pl.BlockSpec((NB, NB), lambda i: (0, 0)),
        out_shape=jax.ShapeDtypeStruct((NB, NB), jnp.float32),
    )(x2, b3)


def _k1b_body(xb, bb, sb, ob):
    i = pl.program_id(0)
    sums = sb[...]
    cnt = jnp.maximum(sums[:, D3:D3 + 1], 1.0)
    mean_x = sums[:, :D3] / cnt
    oh = _onehot(bb[0, 0, :], NB)
    meanb = _dot(oh, mean_x)
    xc = xb[...] - meanb
    nrm = jnp.sqrt(xc[:, 0:K] ** 2 + xc[:, K:2 * K] ** 2 + xc[:, 2 * K:3 * K] ** 2)
    p = _dott(oh, nrm)

    @pl.when(i == 0)
    def _():
        ob[...] = p

    @pl.when(i > 0)
    def _():
        ob[...] = ob[...] + p


def _k1b(x2, b3, sums):
    return pl.pallas_call(
        _k1b_body,
        grid=(NBLK_N,),
        in_specs=[
            pl.BlockSpec((BN, D3), lambda i: (i, 0)),
            pl.BlockSpec((1, 1, BN), lambda i: (i, 0, 0)),
            pl.BlockSpec((NB, NB), lambda i: (0, 0)),
        ],
        out_specs=pl.BlockSpec((NB, K), lambda i: (0, 0)),
        out_shape=jax.ShapeDtypeStruct((NB, K), jnp.float32),
    )(x2, b3, sums)


def _k1c_body(xb, hb, bb, sb, nb_, ab, wb, tso, tto):
    sums = sb[...]
    cnt = jnp.maximum(sums[:, D3:D3 + 1], 1.0)
    mean_x = sums[:, :D3] / cnt
    mn = nb_[...] / cnt
    oh = _onehot(bb[0, 0, :], NB)
    meanb = _dot(oh, mean_x)
    mnb = _dot(oh, mn)
    xc = xb[...] - meanb
    denom = mnb + 1e-5
    den3 = jnp.concatenate([denom, denom, denom], axis=1)
    e3 = ab[2:3, :]
    e33 = jnp.concatenate([e3, e3, e3], axis=1)
    xn = xc * e33 / den3
    h = hb[...]
    mu = jnp.mean(h, axis=1, keepdims=True)
    var = jnp.mean((h - mu) ** 2, axis=1, keepdims=True)
    hn = (h - mu) * lax.rsqrt(var + 1e-5) * ab[0:1, :] + ab[1:2, :]
    gt = _dot(hn, wb[0:HD, :])
    gs = _dot(hn, wb[HD:2 * HD, :])
    tso[...] = jnp.concatenate([xn, gs], axis=1)
    tto[...] = jnp.concatenate([xn, gt], axis=1)


def _k1c(x2, h, b3, sums, nsums, aux, w1):
    return pl.pallas_call(
        _k1c_body,
        grid=(NBLK_N,),
        in_specs=[
            pl.BlockSpec((BN, D3), lambda i: (i, 0)),
            pl.BlockSpec((BN, HD), lambda i: (i, 0)),
            pl.BlockSpec((1, 1, BN), lambda i: (i, 0, 0)),
            pl.BlockSpec((NB, NB), lambda i: (0, 0)),
            pl.BlockSpec((NB, K), lambda i: (0, 0)),
            pl.BlockSpec((8, K), lambda i: (0, 0)),
            pl.BlockSpec((2 * HD + ED + K + HD, HD), lambda i: (0, 0)),
        ],
        out_specs=[
            pl.BlockSpec((BN, D3 + HD), lambda i: (i, 0)),
            pl.BlockSpec((BN, D3 + HD), lambda i: (i, 0)),
        ],
        out_shape=[
            jax.ShapeDtypeStruct((N, D3 + HD), jnp.float32),
            jax.ShapeDtypeStruct((N, D3 + HD), jnp.float32),
        ],
    )(x2, h, b3, sums, nsums, aux, w1)


def _silu(x):
    return x / (1.0 + jnp.exp(-x))


def _k3_body(pb, rb, eb, tb, w1, w2, xw1, xw2, ab, so):
    rd = rb[:, K:2 * K]
    pre = (pb[...] + _dot(eb[...], w1[2 * HD:2 * HD + ED, :])
           + _dot(rd, w1[2 * HD + ED:2 * HD + ED + K, :])
           + _dot(tb[...], w1[2 * HD + ED + K:, :]) + ab[3:4, :])
    h1 = _silu(pre)
    m = _dot(h1, w2[...]) + ab[4:5, :]
    h2 = _silu(_dot(m, xw1[...]) + ab[5:6, :])
    w = jnp.clip(_dot(h2, xw2[...]) + ab[6:7, :], -10.0, 10.0)
    scale = w / (1.0 + jnp.sqrt(rd + 1e-8))
    so[...] = jnp.concatenate([scale, jnp.zeros_like(scale)], axis=1)


def _k3(partial, relpad, ea, te, w1, w2, xw1, xw2, aux):
    return pl.pallas_call(
        _k3_body,
        grid=(NBLK_E,),
        in_specs=[
            pl.BlockSpec((BE, HD), lambda i: (i, 0)),
            pl.BlockSpec((BE, 2 * K), lambda i: (i, 1)),
            pl.BlockSpec((BE, ED), lambda i: (i, 0)),
            pl.BlockSpec((BE, HD), lambda i: (i, 0)),
            pl.BlockSpec((2 * HD + ED + K + HD, HD), lambda i: (0, 0)),
            pl.BlockSpec((HD, HD), lambda i: (0, 0)),
            pl.BlockSpec((HD, HD), lambda i: (0, 0)),
            pl.BlockSpec((HD, K), lambda i: (0, 0)),
            pl.BlockSpec((8, K), lambda i: (0, 0)),
        ],
        out_specs=pl.BlockSpec((BE, 2 * K), lambda i: (i, 0)),
        out_shape=jax.ShapeDtypeStruct((E, 2 * K), jnp.float32),
    )(partial, relpad, ea, te, w1, w2, xw1, xw2, aux)


NCORE = 2
NSUB = 16
NWORK = NCORE * NSUB
EW = E // NWORK
GB = 128
NGB = (EW + GB - 1) // GB

TW = D3 + HD

_K2_MESH = plsc.VectorSubcoreMesh(core_axis_name="c", subcore_axis_name="s")


@functools.partial(
    pl.kernel,
    mesh=_K2_MESH,
    out_type=[
        jax.ShapeDtypeStruct((E, TW), jnp.float32),
        jax.ShapeDtypeStruct((E, HD), jnp.float32),
    ],
    scratch_types=[
        pltpu.VMEM((GB,), jnp.int32),
        pltpu.VMEM((GB,), jnp.int32),
        pltpu.VMEM((GB, TW), jnp.float32),
        pltpu.VMEM((GB, TW), jnp.float32),
        pltpu.VMEM((GB, HD), jnp.float32),
        pltpu.SemaphoreType.DMA,
    ],
)
def _k2(ts_hbm, tt_hbm, src_hbm, tgt_hbm,
        rel_hbm, part_hbm,
        sidx, tidx, sv, tv, pv, sem):
    wid = lax.axis_index("s") * NCORE + lax.axis_index("c")
    ebase = wid * EW

    def blk(b, carry):
        off = ebase + jnp.minimum(b * GB, EW - GB)
        pltpu.sync_copy(src_hbm.at[pl.ds(off, GB)], sidx)
        pltpu.sync_copy(tgt_hbm.at[pl.ds(off, GB)], tidx)
        c1 = pltpu.async_copy(ts_hbm.at[sidx], sv, sem)
        c2 = pltpu.async_copy(tt_hbm.at[tidx], tv, sem)
        c1.wait()
        c2.wait()

        def erow(e, c2_):
            for u in range(HD // 16):
                pv[e, pl.ds(u * 16, 16)] = (tv[e, pl.ds(D3 + u * 16, 16)]
                                            + sv[e, pl.ds(D3 + u * 16, 16)])
            for u in range(D3 // 16):
                sv[e, pl.ds(u * 16, 16)] = sv[e, pl.ds(u * 16, 16)] - tv[e, pl.ds(u * 16, 16)]
            for u in range(K // 16):
                a = sv[e, pl.ds(u * 16, 16)]
                bq = sv[e, pl.ds(K + u * 16, 16)]
                cq = sv[e, pl.ds(2 * K + u * 16, 16)]
                sv[e, pl.ds(D3 + u * 16, 16)] = a * a + bq * bq + cq * cq
            return c2_

        lax.fori_loop(0, GB, erow, 0)
        pltpu.sync_copy(sv, rel_hbm.at[pl.ds(off, GB)])
        pltpu.sync_copy(pv, part_hbm.at[pl.ds(off, GB)])
        return carry

    lax.fori_loop(0, NGB, blk, 0)


def kernel(batch, X, H, edge_index, edge_attr, te, e3_w, ln_w, ln_b,
           pm_w1, pm_b1, pm_w2, pm_b2, px_w1, px_b1, px_w2, px_b2):
    x2 = X.reshape(N, D3)
    b32 = batch.astype(jnp.int32)
    b3 = b32.reshape(NBLK_N, 1, BN)
    src = edge_index[0].astype(jnp.int32)
    tgt = edge_index[1].astype(jnp.int32)

    aux = jnp.stack([ln_w, ln_b, e3_w.reshape(K), pm_b1, pm_b2,
                     px_b1, px_b2, jnp.zeros((K,), jnp.float32)], axis=0)

    sums = _k1a(x2, b3)
    nsums = _k1b(x2, b3, sums)
    ts, tt = _k1c(x2, H, b3, sums, nsums, aux, pm_w1)

    relpad, partial = _k2(ts, tt, src, tgt)

    scalepad = _k3(partial, relpad, edge_attr, te, pm_w1, pm_w2, px_w1, px_w2, aux)

    rel = relpad[:, :D3]
    scale = scalepad[:, :K]
    upd = rel * jnp.concatenate([scale, scale, scale], axis=1)
    out2 = ts[:, :D3] + jax.ops.segment_sum(upd, tgt, num_segments=N)
    return out2.reshape(N, 3, K)
